# Initial kernel scaffold; baseline (speedup 1.0000x reference)
import functools

import jax
import jax.numpy as jnp
from jax import lax
from jax.experimental import pallas as pl
from jax.experimental.pallas import tpu as pltpu
from jax.experimental.pallas import tpu_sc as plsc

N = 10000
E = 320000
D = 128
DW = 16

NC = 2
NS = 16
NW = NC * NS
CHUNK = 128
EPW = 10240
NCHUNK = EPW // CHUNK
E_PAD = NW * EPW
N_PAD = 10016

ZROWS = N_PAD // NS
ZHALF = ZROWS // 2
OROWS = N // NS


def _sc_aggregate(x, srcp, dstp):
    mesh = plsc.VectorSubcoreMesh(core_axis_name="c", subcore_axis_name="s")

    @functools.partial(
        pl.kernel,
        mesh=mesh,
        out_type=(
            jax.ShapeDtypeStruct((NC, N, D), jnp.float32),
            jax.ShapeDtypeStruct((NC, N, DW), jnp.float32),
        ),
        scratch_types=[
            pltpu.VMEM_SHARED((N_PAD, D), jnp.float32),
            pltpu.VMEM_SHARED((N_PAD, DW), jnp.float32),
            pltpu.VMEM((CHUNK,), jnp.int32),
            pltpu.VMEM((CHUNK,), jnp.int32),
            pltpu.VMEM((CHUNK, D), jnp.float32),
            pltpu.VMEM((CHUNK, DW), jnp.float32),
            pltpu.VMEM((ZHALF, D), jnp.float32),
            pltpu.VMEM((ZROWS, DW), jnp.float32),
        ],
    )
    def k(x_hbm, src_hbm, dst_hbm, agg_out, deg_out,
          agg_sp, deg_sp, src_v, dst_v, rows_v, ones_v, zagg_v, zdeg_v):
        c = lax.axis_index("c")
        s = lax.axis_index("s")

        zero16 = jnp.zeros((16,), jnp.float32)
        one16 = jnp.ones((16,), jnp.float32)

        @pl.loop(0, ZHALF)
        def _(i):
            @pl.loop(0, D // 16)
            def _(j):
                zagg_v[i, pl.ds(j * 16, 16)] = zero16

        @pl.loop(0, ZROWS)
        def _(i):
            zdeg_v[i, :] = zero16

        @pl.loop(0, CHUNK)
        def _(i):
            ones_v[i, :] = one16

        zrow = s * ZROWS
        pltpu.sync_copy(zagg_v, agg_sp.at[pl.ds(zrow, ZHALF)])
        pltpu.sync_copy(zagg_v, agg_sp.at[pl.ds(zrow + ZHALF, ZHALF)])
        pltpu.sync_copy(zdeg_v, deg_sp.at[pl.ds(zrow, ZROWS)])
        plsc.subcore_barrier()

        base = (c * NS + s) * EPW

        @pl.loop(0, NCHUNK)
        def _(i):
            off = base + i * CHUNK
            pltpu.sync_copy(src_hbm.at[pl.ds(off, CHUNK)], src_v)
            pltpu.sync_copy(dst_hbm.at[pl.ds(off, CHUNK)], dst_v)
            pltpu.sync_copy(x_hbm.at[src_v], rows_v)
            pltpu.sync_copy(rows_v, agg_sp.at[dst_v], add=True)
            pltpu.sync_copy(ones_v, deg_sp.at[dst_v], add=True)

        plsc.subcore_barrier()

        orow = s * OROWS
        pltpu.sync_copy(agg_sp.at[pl.ds(orow, OROWS)],
                        agg_out.at[c, pl.ds(orow, OROWS)])
        pltpu.sync_copy(deg_sp.at[pl.ds(orow, OROWS)],
                        deg_out.at[c, pl.ds(orow, OROWS)])

    return k(x, srcp, dstp)


BLK = 1000


def _tc_body(x_ref, a0_ref, a1_ref, d0_ref, d1_ref, w_ref, b_ref, o_ref):
    deg = d0_ref[:, 0:1] + d1_ref[:, 0:1]
    inv = 1.0 / jnp.maximum(deg, 1.0)
    nbr = (a0_ref[...] + a1_ref[...]) * inv
    h = jnp.dot(x_ref[...], w_ref[0:D, :], preferred_element_type=jnp.float32)
    h += jnp.dot(nbr, w_ref[D:2 * D, :], preferred_element_type=jnp.float32)
    o_ref[...] = jnp.maximum(h + b_ref[...], 0.0)


def _tc_linear(x, a0, a1, d0, d1, w, b2):
    return pl.pallas_call(
        _tc_body,
        grid=(N // BLK,),
        in_specs=[
            pl.BlockSpec((BLK, D), lambda i: (i, 0)),
            pl.BlockSpec((BLK, D), lambda i: (i, 0)),
            pl.BlockSpec((BLK, D), lambda i: (i, 0)),
            pl.BlockSpec((BLK, DW), lambda i: (i, 0)),
            pl.BlockSpec((BLK, DW), lambda i: (i, 0)),
            pl.BlockSpec((2 * D, D), lambda i: (0, 0)),
            pl.BlockSpec((1, D), lambda i: (0, 0)),
        ],
        out_specs=pl.BlockSpec((BLK, D), lambda i: (i, 0)),
        out_shape=jax.ShapeDtypeStruct((N, D), jnp.float32),
    )(x, a0, a1, d0, d1, w, b2)


def kernel(X, edge_index, W, b):
    src = edge_index[0].astype(jnp.int32)
    dst = edge_index[1].astype(jnp.int32)
    pad = E_PAD - E
    srcp = jnp.concatenate([src, jnp.zeros((pad,), jnp.int32)])
    dstp = jnp.concatenate([dst, jnp.full((pad,), N, jnp.int32)])
    agg, deg = _sc_aggregate(X, srcp, dstp)
    return _tc_linear(X, agg[0], agg[1], deg[0], deg[1], W,
                      b.reshape(1, D))


# speedup vs baseline: 3.5379x; 3.5379x over previous
import functools

import jax
import jax.numpy as jnp
from jax import lax
from jax.experimental import pallas as pl
from jax.experimental.pallas import tpu as pltpu
from jax.experimental.pallas import tpu_sc as plsc

N = 10000
E = 320000
D = 128
DH = D // 2
DW = 16

NC = 2
NS = 16
NW = NC * NS
CHUNK = 128
EPW = 10240
NCHUNK = EPW // CHUNK
E_PAD = NW * EPW
N_PAD = 10240

SHIFT = 14
MASK = (1 << SHIFT) - 1

ZROWS = N_PAD // NS
ZHALF = ZROWS // 2
OROWS = 624
OTAIL = N - NS * OROWS


def _sc_pass(xh, epacked, with_deg):
    mesh = plsc.VectorSubcoreMesh(core_axis_name="c", subcore_axis_name="s")

    out_type = [jax.ShapeDtypeStruct((NC, N, DH), jnp.float32)]
    scratch = [
        pltpu.VMEM_SHARED((N_PAD, DH), jnp.float32),
        pltpu.VMEM((CHUNK,), jnp.int32),
        pltpu.VMEM((CHUNK,), jnp.int32),
        pltpu.VMEM((CHUNK,), jnp.int32),
        pltpu.VMEM((CHUNK, DH), jnp.float32),
        pltpu.VMEM((ZHALF, DH), jnp.float32),
    ]
    if with_deg:
        out_type.append(jax.ShapeDtypeStruct((NC, N, DW), jnp.float32))
        scratch += [
            pltpu.VMEM_SHARED((N_PAD, DW), jnp.float32),
            pltpu.VMEM((CHUNK, DW), jnp.float32),
            pltpu.VMEM((ZROWS, DW), jnp.float32),
        ]

    @functools.partial(
        pl.kernel, mesh=mesh, out_type=tuple(out_type), scratch_types=scratch,
        compiler_params=pltpu.CompilerParams(use_tc_tiling_on_sc=False),
    )
    def k(x_hbm, e_hbm, *refs):
        if with_deg:
            (agg_out, deg_out, agg_sp, pck_v, src_v, dst_v, rows_v, zagg_v,
             deg_sp, ones_v, zdeg_v) = refs
        else:
            agg_out, agg_sp, pck_v, src_v, dst_v, rows_v, zagg_v = refs

        c = lax.axis_index("c")
        s = lax.axis_index("s")

        zero16 = jnp.zeros((16,), jnp.float32)

        @pl.loop(0, ZHALF)
        def _(i):
            @pl.loop(0, DH // 16)
            def _(j):
                zagg_v[i, pl.ds(j * 16, 16)] = zero16

        if with_deg:
            one16 = jnp.ones((16,), jnp.float32)

            @pl.loop(0, ZROWS)
            def _(i):
                zdeg_v[i, :] = zero16

            @pl.loop(0, CHUNK)
            def _(i):
                ones_v[i, :] = one16

        zrow = s * ZROWS
        pltpu.sync_copy(zagg_v, agg_sp.at[pl.ds(zrow, ZHALF)])
        pltpu.sync_copy(zagg_v, agg_sp.at[pl.ds(zrow + ZHALF, ZHALF)])
        if with_deg:
            pltpu.sync_copy(zdeg_v, deg_sp.at[pl.ds(zrow, ZROWS)])
        plsc.subcore_barrier()

        base = (c * NS + s) * EPW

        @pl.loop(0, NCHUNK)
        def _(i):
            off = base + i * CHUNK
            pltpu.sync_copy(e_hbm.at[pl.ds(off, CHUNK)], pck_v)

            @pl.loop(0, CHUNK // 16)
            def _(j):
                p = pck_v[pl.ds(j * 16, 16)]
                src_v[pl.ds(j * 16, 16)] = lax.shift_right_logical(p, SHIFT)
                dst_v[pl.ds(j * 16, 16)] = lax.bitwise_and(p, MASK)

            pltpu.sync_copy(x_hbm.at[src_v], rows_v)
            pltpu.sync_copy(rows_v, agg_sp.at[dst_v], add=True)
            if with_deg:
                pltpu.sync_copy(ones_v, deg_sp.at[dst_v], add=True)

        plsc.subcore_barrier()

        orow = s * OROWS
        pltpu.sync_copy(agg_sp.at[pl.ds(orow, OROWS)],
                        agg_out.at[c, pl.ds(orow, OROWS)])
        if with_deg:
            pltpu.sync_copy(deg_sp.at[pl.ds(orow, OROWS)],
                            deg_out.at[c, pl.ds(orow, OROWS)])

        @pl.when(s == NS - 1)
        def _():
            tail = NS * OROWS
            pltpu.sync_copy(agg_sp.at[pl.ds(tail, OTAIL)],
                            agg_out.at[c, pl.ds(tail, OTAIL)])
            if with_deg:
                pltpu.sync_copy(deg_sp.at[pl.ds(tail, OTAIL)],
                                deg_out.at[c, pl.ds(tail, OTAIL)])

    return k(xh, epacked)


BLK = 1000


def _tc_body(x_ref, l0_ref, l1_ref, r0_ref, r1_ref, d0_ref, d1_ref,
             w_ref, b_ref, o_ref):
    deg = d0_ref[0, :, 0:1] + d1_ref[0, :, 0:1]
    inv = 1.0 / jnp.maximum(deg, 1.0)
    nbr_l = (l0_ref[0] + l1_ref[0]) * inv
    nbr_r = (r0_ref[0] + r1_ref[0]) * inv
    h = jnp.dot(x_ref[...], w_ref[0:D, :], preferred_element_type=jnp.float32)
    h += jnp.dot(nbr_l, w_ref[D:D + DH, :],
                 preferred_element_type=jnp.float32)
    h += jnp.dot(nbr_r, w_ref[D + DH:2 * D, :],
                 preferred_element_type=jnp.float32)
    o_ref[...] = jnp.maximum(h + b_ref[...], 0.0)


def _tc_linear(x, aggl, aggr, deg, w, b2):
    blk_d = pl.BlockSpec((BLK, D), lambda i: (i, 0))
    blk_h = pl.BlockSpec((1, BLK, DH), lambda i: (0, i, 0))
    blk_h1 = pl.BlockSpec((1, BLK, DH), lambda i: (1, i, 0))
    blk_w = pl.BlockSpec((1, BLK, DW), lambda i: (0, i, 0))
    blk_w1 = pl.BlockSpec((1, BLK, DW), lambda i: (1, i, 0))
    return pl.pallas_call(
        _tc_body,
        grid=(N // BLK,),
        in_specs=[blk_d, blk_h, blk_h1, blk_h, blk_h1, blk_w, blk_w1,
                  pl.BlockSpec((2 * D, D), lambda i: (0, 0)),
                  pl.BlockSpec((1, D), lambda i: (0, 0))],
        out_specs=blk_d,
        out_shape=jax.ShapeDtypeStruct((N, D), jnp.float32),
    )(x, aggl, aggl, aggr, aggr, deg, deg, w, b2)


def kernel(X, edge_index, W, b):
    src = edge_index[0].astype(jnp.int32)
    dst = edge_index[1].astype(jnp.int32)
    pad = E_PAD - E
    packed = jnp.concatenate([
        jnp.left_shift(src, SHIFT) | dst,
        jnp.full((pad,), N, jnp.int32),
    ])
    xl = X[:, :DH]
    xr = X[:, DH:]
    aggl, deg = _sc_pass(xl, packed, True)
    (aggr,) = _sc_pass(xr, packed, False)
    return _tc_linear(X, aggl, aggr, deg, W, b.reshape(1, D))


# speedup vs baseline: 4.8180x; 1.3618x over previous
import functools

import jax
import jax.numpy as jnp
from jax import lax
from jax.experimental import pallas as pl
from jax.experimental.pallas import tpu as pltpu
from jax.experimental.pallas import tpu_sc as plsc

N = 10000
E = 320000
D = 128
DH = D // 2
DW = 16

NC = 2
NS = 16
NW = NC * NS
CHUNK = 128
EPW = 10240
NCHUNK = EPW // CHUNK
E_PAD = NW * EPW
N_PAD = 10240

SHIFT = 14
MASK = (1 << SHIFT) - 1

ZROWS = N_PAD // NS
ZHALF = ZROWS // 2
OROWS = 624
OTAIL = N - NS * OROWS


def _sc_pass(xh, epacked, with_deg):
    mesh = plsc.VectorSubcoreMesh(core_axis_name="c", subcore_axis_name="s")

    out_type = [jax.ShapeDtypeStruct((NC, N, DH), jnp.float32)]
    scratch = [
        pltpu.VMEM_SHARED((N_PAD, DH), jnp.float32),
        pltpu.VMEM((2, CHUNK), jnp.int32),
        pltpu.VMEM((2, CHUNK), jnp.int32),
        pltpu.VMEM((2, CHUNK), jnp.int32),
        pltpu.VMEM((2, CHUNK, DH), jnp.float32),
        pltpu.VMEM((ZHALF, DH), jnp.float32),
        pltpu.SemaphoreType.DMA,
        pltpu.SemaphoreType.DMA,
        pltpu.SemaphoreType.DMA,
        pltpu.SemaphoreType.DMA,
        pltpu.SemaphoreType.DMA,
        pltpu.SemaphoreType.DMA,
    ]
    if with_deg:
        out_type.append(jax.ShapeDtypeStruct((NC, N, DW), jnp.float32))
        scratch += [
            pltpu.VMEM_SHARED((N_PAD, DW), jnp.float32),
            pltpu.VMEM((CHUNK, DW), jnp.float32),
            pltpu.VMEM((ZROWS, DW), jnp.float32),
            pltpu.SemaphoreType.DMA,
            pltpu.SemaphoreType.DMA,
        ]

    @functools.partial(
        pl.kernel, mesh=mesh, out_type=tuple(out_type), scratch_types=scratch,
        compiler_params=pltpu.CompilerParams(use_tc_tiling_on_sc=False),
    )
    def k(x_hbm, e_hbm, *refs):
        if with_deg:
            (agg_out, deg_out, agg_sp, pck_v, src_v, dst_v, rows_v, zagg_v,
             si0, si1, sg0, sg1, sa0, sa1,
             deg_sp, ones_v, zdeg_v, sd0, sd1) = refs
            sd = (sd0, sd1)
        else:
            (agg_out, agg_sp, pck_v, src_v, dst_v, rows_v, zagg_v,
             si0, si1, sg0, sg1, sa0, sa1) = refs
        si = (si0, si1)
        sg = (sg0, sg1)
        sa = (sa0, sa1)

        c = lax.axis_index("c")
        s = lax.axis_index("s")

        zero16 = jnp.zeros((16,), jnp.float32)

        @pl.loop(0, ZHALF)
        def _(i):
            @pl.loop(0, DH // 16)
            def _(j):
                zagg_v[i, pl.ds(j * 16, 16)] = zero16

        if with_deg:
            one16 = jnp.ones((16,), jnp.float32)

            @pl.loop(0, ZROWS)
            def _(i):
                zdeg_v[i, :] = zero16

            @pl.loop(0, CHUNK)
            def _(i):
                ones_v[i, :] = one16

        zrow = s * ZROWS
        pltpu.sync_copy(zagg_v, agg_sp.at[pl.ds(zrow, ZHALF)])
        pltpu.sync_copy(zagg_v, agg_sp.at[pl.ds(zrow + ZHALF, ZHALF)])
        if with_deg:
            pltpu.sync_copy(zdeg_v, deg_sp.at[pl.ds(zrow, ZROWS)])
        plsc.subcore_barrier()

        base = (c * NS + s) * EPW

        def issue_idx(i, k):
            pltpu.async_copy(e_hbm.at[pl.ds(base + i * CHUNK, CHUNK)],
                             pck_v.at[k], si[k])

        def wait_idx(k):
            pltpu.make_async_copy(e_hbm.at[pl.ds(0, CHUNK)],
                                  pck_v.at[k], si[k]).wait()

        def unpack(k):
            @pl.loop(0, CHUNK // 16)
            def _(j):
                p = pck_v[k, pl.ds(j * 16, 16)]
                src_v[k, pl.ds(j * 16, 16)] = lax.shift_right_logical(p, SHIFT)
                dst_v[k, pl.ds(j * 16, 16)] = lax.bitwise_and(p, MASK)

        def issue_gather(k):
            pltpu.async_copy(x_hbm.at[src_v.at[k]], rows_v.at[k], sg[k])

        def wait_gather(k):
            pltpu.make_async_copy(x_hbm.at[src_v.at[k]],
                                  rows_v.at[k], sg[k]).wait()

        def issue_scatter(k):
            pltpu.async_copy(rows_v.at[k], agg_sp.at[dst_v.at[k]], sa[k],
                             add=True)
            if with_deg:
                pltpu.async_copy(ones_v, deg_sp.at[dst_v.at[k]], sd[k],
                                 add=True)

        def wait_scatter(k):
            pltpu.make_async_copy(rows_v.at[k], agg_sp.at[dst_v.at[k]],
                                  sa[k]).wait()
            if with_deg:
                pltpu.make_async_copy(ones_v, deg_sp.at[dst_v.at[k]],
                                      sd[k]).wait()

        pltpu.sync_copy(e_hbm.at[pl.ds(base, CHUNK)], pck_v.at[0])
        unpack(0)
        issue_gather(0)
        issue_idx(1, 1)

        @pl.loop(0, NCHUNK // 2)
        def _(g):
            for k in (0, 1):
                nxt = 1 - k
                def prep():
                    wait_idx(nxt)

                    def drain():
                        wait_scatter(nxt)
                    if k == 0:
                        pl.when(g > 0)(drain)
                    else:
                        drain()
                    unpack(nxt)

                    def fetch():
                        issue_idx(2 * g + k + 2, k)
                    pl.when(g < NCHUNK // 2 - 1)(fetch)
                    issue_gather(nxt)
                if k == 0:
                    prep()
                else:
                    pl.when(g < NCHUNK // 2 - 1)(prep)
                wait_gather(k)
                issue_scatter(k)

        wait_scatter(0)
        wait_scatter(1)
        plsc.subcore_barrier()

        orow = s * OROWS
        pltpu.sync_copy(agg_sp.at[pl.ds(orow, OROWS)],
                        agg_out.at[c, pl.ds(orow, OROWS)])
        if with_deg:
            pltpu.sync_copy(deg_sp.at[pl.ds(orow, OROWS)],
                            deg_out.at[c, pl.ds(orow, OROWS)])

        @pl.when(s == NS - 1)
        def _():
            tail = NS * OROWS
            pltpu.sync_copy(agg_sp.at[pl.ds(tail, OTAIL)],
                            agg_out.at[c, pl.ds(tail, OTAIL)])
            if with_deg:
                pltpu.sync_copy(deg_sp.at[pl.ds(tail, OTAIL)],
                                deg_out.at[c, pl.ds(tail, OTAIL)])

    return k(xh, epacked)


BLK = 1000


def _tc_body(x_ref, l0_ref, l1_ref, r0_ref, r1_ref, d0_ref, d1_ref,
             w_ref, b_ref, o_ref):
    deg = d0_ref[0, :, 0:1] + d1_ref[0, :, 0:1]
    inv = 1.0 / jnp.maximum(deg, 1.0)
    nbr_l = (l0_ref[0] + l1_ref[0]) * inv
    nbr_r = (r0_ref[0] + r1_ref[0]) * inv
    h = jnp.dot(x_ref[...], w_ref[0:D, :], preferred_element_type=jnp.float32)
    h += jnp.dot(nbr_l, w_ref[D:D + DH, :],
                 preferred_element_type=jnp.float32)
    h += jnp.dot(nbr_r, w_ref[D + DH:2 * D, :],
                 preferred_element_type=jnp.float32)
    o_ref[...] = jnp.maximum(h + b_ref[...], 0.0)


def _tc_linear(x, aggl, aggr, deg, w, b2):
    blk_d = pl.BlockSpec((BLK, D), lambda i: (i, 0))
    blk_h = pl.BlockSpec((1, BLK, DH), lambda i: (0, i, 0))
    blk_h1 = pl.BlockSpec((1, BLK, DH), lambda i: (1, i, 0))
    blk_w = pl.BlockSpec((1, BLK, DW), lambda i: (0, i, 0))
    blk_w1 = pl.BlockSpec((1, BLK, DW), lambda i: (1, i, 0))
    return pl.pallas_call(
        _tc_body,
        grid=(N // BLK,),
        in_specs=[blk_d, blk_h, blk_h1, blk_h, blk_h1, blk_w, blk_w1,
                  pl.BlockSpec((2 * D, D), lambda i: (0, 0)),
                  pl.BlockSpec((1, D), lambda i: (0, 0))],
        out_specs=blk_d,
        out_shape=jax.ShapeDtypeStruct((N, D), jnp.float32),
    )(x, aggl, aggl, aggr, aggr, deg, deg, w, b2)


def kernel(X, edge_index, W, b):
    src = edge_index[0].astype(jnp.int32)
    dst = edge_index[1].astype(jnp.int32)
    pad = E_PAD - E
    packed = jnp.concatenate([
        jnp.left_shift(src, SHIFT) | dst,
        jnp.full((pad,), N, jnp.int32),
    ])
    xl = X[:, :DH]
    xr = X[:, DH:]
    aggl, deg = _sc_pass(xl, packed, True)
    (aggr,) = _sc_pass(xr, packed, False)
    return _tc_linear(X, aggl, aggr, deg, W, b.reshape(1, D))


# speedup vs baseline: 5.7092x; 1.1850x over previous
import functools

import jax
import jax.numpy as jnp
from jax import lax
from jax.experimental import pallas as pl
from jax.experimental.pallas import tpu as pltpu
from jax.experimental.pallas import tpu_sc as plsc

N = 10000
E = 320000
D = 128
DH = D // 2
DW = 16

NC = 2
NS = 16
CHUNK = 128
G = 2
GLEN = G * CHUNK
EPW = 20480
NG = EPW // GLEN
E_PAD = NS * EPW
N_PAD = 10240

SHIFT = 14
MASK = (1 << SHIFT) - 1

ZROWS = N_PAD // NS
ZHALF = ZROWS // 2
OROWS = 624
OTAIL = N - NS * OROWS


def _sc_aggregate(xl, xr, epacked):
    mesh = plsc.VectorSubcoreMesh(core_axis_name="c", subcore_axis_name="s")

    @functools.partial(
        pl.kernel, mesh=mesh,
        out_type=(
            jax.ShapeDtypeStruct((N, DH), jnp.float32),
            jax.ShapeDtypeStruct((N, DH), jnp.float32),
            jax.ShapeDtypeStruct((N, DW), jnp.float32),
        ),
        scratch_types=[
            pltpu.VMEM_SHARED((N_PAD, DH), jnp.float32),
            pltpu.VMEM_SHARED((N_PAD, DW), jnp.float32),
            pltpu.VMEM((2, GLEN), jnp.int32),
            pltpu.VMEM((2, G, CHUNK), jnp.int32),
            pltpu.VMEM((2, G, CHUNK), jnp.int32),
            pltpu.VMEM((2, G, CHUNK, DH), jnp.float32),
            pltpu.VMEM((CHUNK, DW), jnp.float32),
            pltpu.VMEM((ZHALF, DH), jnp.float32),
            pltpu.VMEM((ZROWS, DW), jnp.float32),
            pltpu.SemaphoreType.DMA,
            pltpu.SemaphoreType.DMA,
            pltpu.SemaphoreType.DMA,
            pltpu.SemaphoreType.DMA,
            pltpu.SemaphoreType.DMA,
            pltpu.SemaphoreType.DMA,
            pltpu.SemaphoreType.DMA,
            pltpu.SemaphoreType.DMA,
        ],
        compiler_params=pltpu.CompilerParams(use_tc_tiling_on_sc=False),
    )
    def k(xl_hbm, xr_hbm, e_hbm, aggl_out, aggr_out, deg_out,
          agg_sp, deg_sp, pck_v, src_v, dst_v, rows_v, ones_v,
          zagg_v, zdeg_v, si0, si1, sg0, sg1, sa0, sa1, sd0, sd1):
        si = (si0, si1)
        sg = (sg0, sg1)
        sa = (sa0, sa1)
        sd = (sd0, sd1)

        c = lax.axis_index("c")
        s = lax.axis_index("s")
        on_sc0 = c == 0

        zero16 = jnp.zeros((16,), jnp.float32)
        one16 = jnp.ones((16,), jnp.float32)

        @pl.loop(0, ZHALF)
        def _(i):
            @pl.loop(0, DH // 16)
            def _(j):
                zagg_v[i, pl.ds(j * 16, 16)] = zero16

        @pl.when(on_sc0)
        def _():
            @pl.loop(0, ZROWS)
            def _(i):
                zdeg_v[i, :] = zero16

            @pl.loop(0, CHUNK)
            def _(i):
                ones_v[i, :] = one16

        zrow = s * ZROWS
        pltpu.sync_copy(zagg_v, agg_sp.at[pl.ds(zrow, ZHALF)])
        pltpu.sync_copy(zagg_v, agg_sp.at[pl.ds(zrow + ZHALF, ZHALF)])

        @pl.when(on_sc0)
        def _():
            pltpu.sync_copy(zdeg_v, deg_sp.at[pl.ds(zrow, ZROWS)])
        plsc.subcore_barrier()

        base = s * EPW

        def issue_idx(g, b):
            pltpu.async_copy(e_hbm.at[pl.ds(base + g * GLEN, GLEN)],
                             pck_v.at[b], si[b])

        def wait_idx(b):
            pltpu.make_async_copy(e_hbm.at[pl.ds(0, GLEN)],
                                  pck_v.at[b], si[b]).wait()

        def unpack(b):
            for j in range(G):
                @pl.loop(0, CHUNK // 16)
                def _(u):
                    p = pck_v[b, pl.ds(j * CHUNK + u * 16, 16)]
                    src_v[b, j, pl.ds(u * 16, 16)] = (
                        lax.shift_right_logical(p, SHIFT))
                    dst_v[b, j, pl.ds(u * 16, 16)] = lax.bitwise_and(p, MASK)

        def issue_gathers(b):
            for j in range(G):
                @pl.when(on_sc0)
                def _():
                    pltpu.async_copy(xl_hbm.at[src_v.at[b, j]],
                                     rows_v.at[b, j], sg[b])

                @pl.when(~on_sc0)
                def _():
                    pltpu.async_copy(xr_hbm.at[src_v.at[b, j]],
                                     rows_v.at[b, j], sg[b])

        def wait_gathers(b):
            for j in range(G):
                pltpu.make_async_copy(xl_hbm.at[src_v.at[b, j]],
                                      rows_v.at[b, j], sg[b]).wait()

        def issue_scatters(b):
            for j in range(G):
                pltpu.async_copy(rows_v.at[b, j], agg_sp.at[dst_v.at[b, j]],
                                 sa[b], add=True)

            @pl.when(on_sc0)
            def _():
                for j in range(G):
                    pltpu.async_copy(ones_v, deg_sp.at[dst_v.at[b, j]],
                                     sd[b], add=True)

        def wait_scatters(b):
            for j in range(G):
                pltpu.make_async_copy(rows_v.at[b, j],
                                      agg_sp.at[dst_v.at[b, j]], sa[b]).wait()

            @pl.when(on_sc0)
            def _():
                for j in range(G):
                    pltpu.make_async_copy(ones_v, deg_sp.at[dst_v.at[b, j]],
                                          sd[b]).wait()

        pltpu.sync_copy(e_hbm.at[pl.ds(base, GLEN)], pck_v.at[0])
        unpack(0)
        issue_gathers(0)
        issue_idx(1, 1)

        @pl.loop(0, NG // 2)
        def _(t):
            for b in (0, 1):
                nxt = 1 - b

                def prep():
                    wait_idx(nxt)

                    def drain():
                        wait_scatters(nxt)
                    if b == 0:
                        pl.when(t > 0)(drain)
                    else:
                        drain()
                    unpack(nxt)

                    def fetch():
                        issue_idx(2 * t + b + 2, b)
                    if b == 0:
                        pl.when(t < NG // 2 - 1)(fetch)
                    else:
                        fetch()
                    issue_gathers(nxt)
                if b == 0:
                    prep()
                else:
                    pl.when(t < NG // 2 - 1)(prep)
                wait_gathers(b)
                issue_scatters(b)

        wait_scatters(0)
        wait_scatters(1)
        plsc.subcore_barrier()

        orow = s * OROWS

        def writeback(agg_dst):
            pltpu.sync_copy(agg_sp.at[pl.ds(orow, OROWS)],
                            agg_dst.at[pl.ds(orow, OROWS)])

            @pl.when(s == NS - 1)
            def _():
                tail = NS * OROWS
                pltpu.sync_copy(agg_sp.at[pl.ds(tail, OTAIL)],
                                agg_dst.at[pl.ds(tail, OTAIL)])

        @pl.when(on_sc0)
        def _():
            writeback(aggl_out)
            pltpu.sync_copy(deg_sp.at[pl.ds(orow, OROWS)],
                            deg_out.at[pl.ds(orow, OROWS)])

            @pl.when(s == NS - 1)
            def _():
                tail = NS * OROWS
                pltpu.sync_copy(deg_sp.at[pl.ds(tail, OTAIL)],
                                deg_out.at[pl.ds(tail, OTAIL)])

        @pl.when(~on_sc0)
        def _():
            writeback(aggr_out)

    return k(xl, xr, epacked)


BLK = 1000


def _tc_body(x_ref, al_ref, ar_ref, d_ref, w_ref, b_ref, o_ref):
    inv = 1.0 / jnp.maximum(d_ref[:, 0:1], 1.0)
    h = jnp.dot(x_ref[...], w_ref[0:D, :], preferred_element_type=jnp.float32)
    h += jnp.dot(al_ref[...] * inv, w_ref[D:D + DH, :],
                 preferred_element_type=jnp.float32)
    h += jnp.dot(ar_ref[...] * inv, w_ref[D + DH:2 * D, :],
                 preferred_element_type=jnp.float32)
    o_ref[...] = jnp.maximum(h + b_ref[...], 0.0)


def _tc_linear(x, aggl, aggr, deg, w, b2):
    return pl.pallas_call(
        _tc_body,
        grid=(N // BLK,),
        in_specs=[
            pl.BlockSpec((BLK, D), lambda i: (i, 0)),
            pl.BlockSpec((BLK, DH), lambda i: (i, 0)),
            pl.BlockSpec((BLK, DH), lambda i: (i, 0)),
            pl.BlockSpec((BLK, DW), lambda i: (i, 0)),
            pl.BlockSpec((2 * D, D), lambda i: (0, 0)),
            pl.BlockSpec((1, D), lambda i: (0, 0)),
        ],
        out_specs=pl.BlockSpec((BLK, D), lambda i: (i, 0)),
        out_shape=jax.ShapeDtypeStruct((N, D), jnp.float32),
    )(x, aggl, aggr, deg, w, b2)


def kernel(X, edge_index, W, b):
    src = edge_index[0].astype(jnp.int32)
    dst = edge_index[1].astype(jnp.int32)
    pad = E_PAD - E
    packed = jnp.concatenate([
        jnp.left_shift(src, SHIFT) | dst,
        jnp.full((pad,), N, jnp.int32),
    ])
    xl = X[:, :DH]
    xr = X[:, DH:]
    aggl, aggr, deg = _sc_aggregate(xl, xr, packed)
    return _tc_linear(X, aggl, aggr, deg, W, b.reshape(1, D))


# speedup vs baseline: 5.7543x; 1.0079x over previous
import functools

import jax
import jax.numpy as jnp
from jax import lax
from jax.experimental import pallas as pl
from jax.experimental.pallas import tpu as pltpu
from jax.experimental.pallas import tpu_sc as plsc

N = 10000
E = 320000
D = 128
DH = D // 2
DW = 16
AW = DH + DW

NC = 2
NS = 16
CHUNK = 128
G = 2
GLEN = G * CHUNK
EPW = 20480
NG = EPW // GLEN
E_PAD = NS * EPW
N_PAD = 10240

SHIFT = 14
MASK = (1 << SHIFT) - 1

ZROWS = N_PAD // NS
ZHALF = ZROWS // 2
OROWS = 624
OTAIL = N - NS * OROWS


def _sc_aggregate(xal, xar, epacked):
    mesh = plsc.VectorSubcoreMesh(core_axis_name="c", subcore_axis_name="s")

    @functools.partial(
        pl.kernel, mesh=mesh,
        out_type=(
            jax.ShapeDtypeStruct((N, AW), jnp.float32),
            jax.ShapeDtypeStruct((N, AW), jnp.float32),
        ),
        scratch_types=[
            pltpu.VMEM_SHARED((N_PAD, AW), jnp.float32),
            pltpu.VMEM((2, GLEN), jnp.int32),
            pltpu.VMEM((2, G, CHUNK), jnp.int32),
            pltpu.VMEM((2, G, CHUNK), jnp.int32),
            pltpu.VMEM((2, G, CHUNK, AW), jnp.float32),
            pltpu.VMEM((ZHALF, AW), jnp.float32),
            pltpu.SemaphoreType.DMA,
            pltpu.SemaphoreType.DMA,
            pltpu.SemaphoreType.DMA,
            pltpu.SemaphoreType.DMA,
            pltpu.SemaphoreType.DMA,
            pltpu.SemaphoreType.DMA,
        ],
        compiler_params=pltpu.CompilerParams(use_tc_tiling_on_sc=False),
    )
    def k(xal_hbm, xar_hbm, e_hbm, aggl_out, aggr_out,
          agg_sp, pck_v, src_v, dst_v, rows_v, zagg_v,
          si0, si1, sg0, sg1, sa0, sa1):
        si = (si0, si1)
        sg = (sg0, sg1)
        sa = (sa0, sa1)

        c = lax.axis_index("c")
        s = lax.axis_index("s")
        on_sc0 = c == 0

        zero16 = jnp.zeros((16,), jnp.float32)

        @pl.loop(0, ZHALF)
        def _(i):
            @pl.loop(0, AW // 16)
            def _(j):
                zagg_v[i, pl.ds(j * 16, 16)] = zero16

        zrow = s * ZROWS
        pltpu.sync_copy(zagg_v, agg_sp.at[pl.ds(zrow, ZHALF)])
        pltpu.sync_copy(zagg_v, agg_sp.at[pl.ds(zrow + ZHALF, ZHALF)])
        plsc.subcore_barrier()

        base = s * EPW

        def issue_idx(g, b):
            pltpu.async_copy(e_hbm.at[pl.ds(base + g * GLEN, GLEN)],
                             pck_v.at[b], si[b])

        def wait_idx(b):
            pltpu.make_async_copy(e_hbm.at[pl.ds(0, GLEN)],
                                  pck_v.at[b], si[b]).wait()

        def unpack(b):
            for j in range(G):
                @pl.loop(0, CHUNK // 16)
                def _(u):
                    p = pck_v[b, pl.ds(j * CHUNK + u * 16, 16)]
                    src_v[b, j, pl.ds(u * 16, 16)] = (
                        lax.shift_right_logical(p, SHIFT))
                    dst_v[b, j, pl.ds(u * 16, 16)] = lax.bitwise_and(p, MASK)

        def issue_gathers(b):
            @pl.when(on_sc0)
            def _():
                @pl.loop(0, G)
                def _(j):
                    pltpu.async_copy(xal_hbm.at[src_v.at[b, j]],
                                     rows_v.at[b, j], sg[b])

            @pl.when(~on_sc0)
            def _():
                @pl.loop(0, G)
                def _(j):
                    pltpu.async_copy(xar_hbm.at[src_v.at[b, j]],
                                     rows_v.at[b, j], sg[b])

        def wait_gathers(b):
            @pl.loop(0, G)
            def _(j):
                pltpu.make_async_copy(xal_hbm.at[src_v.at[b, j]],
                                      rows_v.at[b, j], sg[b]).wait()

        def issue_scatters(b):
            @pl.loop(0, G)
            def _(j):
                pltpu.async_copy(rows_v.at[b, j], agg_sp.at[dst_v.at[b, j]],
                                 sa[b], add=True)

        def wait_scatters(b):
            @pl.loop(0, G)
            def _(j):
                pltpu.make_async_copy(rows_v.at[b, j],
                                      agg_sp.at[dst_v.at[b, j]], sa[b]).wait()

        pltpu.sync_copy(e_hbm.at[pl.ds(base, GLEN)], pck_v.at[0])
        unpack(0)
        issue_gathers(0)
        issue_idx(1, 1)

        @pl.loop(0, NG // 2)
        def _(t):
            for b in (0, 1):
                nxt = 1 - b

                def prep():
                    wait_idx(nxt)

                    def drain():
                        wait_scatters(nxt)
                    if b == 0:
                        pl.when(t > 0)(drain)
                    else:
                        drain()
                    unpack(nxt)

                    def fetch():
                        issue_idx(2 * t + b + 2, b)
                    if b == 0:
                        pl.when(t < NG // 2 - 1)(fetch)
                    else:
                        fetch()
                    issue_gathers(nxt)
                if b == 0:
                    prep()
                else:
                    pl.when(t < NG // 2 - 1)(prep)
                wait_gathers(b)
                issue_scatters(b)

        wait_scatters(0)
        wait_scatters(1)
        plsc.subcore_barrier()

        orow = s * OROWS

        def writeback(agg_dst):
            pltpu.sync_copy(agg_sp.at[pl.ds(orow, OROWS)],
                            agg_dst.at[pl.ds(orow, OROWS)])

            @pl.when(s == NS - 1)
            def _():
                tail = NS * OROWS
                pltpu.sync_copy(agg_sp.at[pl.ds(tail, OTAIL)],
                                agg_dst.at[pl.ds(tail, OTAIL)])

        @pl.when(on_sc0)
        def _():
            writeback(aggl_out)

        @pl.when(~on_sc0)
        def _():
            writeback(aggr_out)

    return k(xal, xar, epacked)


BLK = 1000


def _tc_body(x_ref, al_ref, ar_ref, w_ref, b_ref, o_ref):
    inv = 1.0 / jnp.maximum(al_ref[:, DH:DH + 1], 1.0)
    h = jnp.dot(x_ref[...], w_ref[0:D, :], preferred_element_type=jnp.float32)
    h += jnp.dot(al_ref[:, :DH] * inv, w_ref[D:D + DH, :],
                 preferred_element_type=jnp.float32)
    h += jnp.dot(ar_ref[:, :DH] * inv, w_ref[D + DH:2 * D, :],
                 preferred_element_type=jnp.float32)
    o_ref[...] = jnp.maximum(h + b_ref[...], 0.0)


def _tc_linear(x, aggl, aggr, w, b2):
    return pl.pallas_call(
        _tc_body,
        grid=(N // BLK,),
        in_specs=[
            pl.BlockSpec((BLK, D), lambda i: (i, 0)),
            pl.BlockSpec((BLK, AW), lambda i: (i, 0)),
            pl.BlockSpec((BLK, AW), lambda i: (i, 0)),
            pl.BlockSpec((2 * D, D), lambda i: (0, 0)),
            pl.BlockSpec((1, D), lambda i: (0, 0)),
        ],
        out_specs=pl.BlockSpec((BLK, D), lambda i: (i, 0)),
        out_shape=jax.ShapeDtypeStruct((N, D), jnp.float32),
    )(x, aggl, aggr, w, b2)


def kernel(X, edge_index, W, b):
    src = edge_index[0].astype(jnp.int32)
    dst = edge_index[1].astype(jnp.int32)
    pad = E_PAD - E
    packed = jnp.concatenate([
        jnp.left_shift(src, SHIFT) | dst,
        jnp.full((pad,), N, jnp.int32),
    ])
    ones = jnp.ones((N, DW), jnp.float32)
    xal = jnp.concatenate([X[:, :DH], ones], axis=1)
    xar = jnp.concatenate([X[:, DH:], ones], axis=1)
    aggl, aggr = _sc_aggregate(xal, xar, packed)
    return _tc_linear(X, aggl, aggr, W, b.reshape(1, D))
